# baseline (speedup 1.0000x reference)
import functools

import jax
import jax.numpy as jnp
import numpy as np
from jax import lax
from jax.experimental import pallas as pl
from jax.experimental.pallas import tpu as pltpu
from jax.experimental.pallas import tpu_sc as plsc

B = 16384
D = 128
NC = 2
NS = 16
NW = NC * NS
BPW = B // NW
CH = 128
NSTREAM = BPW // CH
BLK = 4096

_PERM = np.empty((D,), np.int64)
for _g in range(D // 32):
    for _i in range(16):
        _PERM[32 * _g + 2 * _i] = 32 * _g + _i
        _PERM[32 * _g + 2 * _i + 1] = 32 * _g + 16 + _i


def _gather_sc(idx2d, emb_table):
    mesh = plsc.VectorSubcoreMesh(core_axis_name="c", subcore_axis_name="s")

    @functools.partial(
        pl.kernel,
        mesh=mesh,
        out_type=jax.ShapeDtypeStruct((B, D // 2), jnp.int32),
        scratch_types=[
            pltpu.VMEM((NSTREAM, CH), jnp.int32),
            pltpu.VMEM((2, CH, D), jnp.int32),
            pltpu.VMEM((2, CH, D // 2), jnp.int32),
            pltpu.SemaphoreType.DMA,
            pltpu.SemaphoreType.DMA,
        ],
    )
    def k(idx_hbm, table_hbm, out_hbm, idx_v, rows_v, pw_v, sem, osem):
        wid = lax.axis_index("s") * NC + lax.axis_index("c")
        pltpu.sync_copy(idx_hbm.at[pl.ds(wid * NSTREAM, NSTREAM)], idx_v)

        def gather(j):
            return pltpu.async_copy(
                table_hbm.at[idx_v.at[j]], rows_v.at[j % 2], sem
            )

        copies = {0: gather(0), 1: gather(1)}
        outs = {}
        half = jnp.full((16,), 0x8000, jnp.int32)
        himask = jnp.full((16,), -65536, jnp.int32)
        for j in range(NSTREAM):
            copies[j].wait()
            if j >= 2:
                outs[j - 2].wait()
            buf = j % 2

            def pack_row(r, _):
                for g in range(D // 32):
                    a = rows_v[buf, r, pl.ds(32 * g, 16)]
                    b = rows_v[buf, r, pl.ds(32 * g + 16, 16)]
                    w = lax.shift_right_logical(a + half, 16) | (
                        (b + half) & himask)
                    pw_v[buf, r, pl.ds(16 * g, 16)] = w
                return ()

            lax.fori_loop(0, CH, pack_row, ())
            outs[j] = pltpu.async_copy(
                pw_v.at[buf],
                out_hbm.at[pl.ds(wid * BPW + j * CH, CH)],
                osem,
            )
            if j + 2 < NSTREAM:
                copies[j + 2] = gather(j + 2)
        outs[NSTREAM - 2].wait()
        outs[NSTREAM - 1].wait()

    return k(idx2d, emb_table)


def _mlp_body(x_ref, w1_ref, s1_ref, c1_ref, w2_ref, b2_ref, o_ref):
    x = x_ref[...].astype(jnp.float32)
    m = jnp.mean(x, axis=-1, keepdims=True)
    q = jnp.mean(x * x, axis=-1, keepdims=True)
    rstd = lax.rsqrt(q - m * m + 1e-5)
    p = jnp.dot(x, w1_ref[...], preferred_element_type=jnp.float32)
    h = rstd * (p - m * s1_ref[...]) + c1_ref[...]
    h = h * jax.nn.sigmoid(h)
    o_ref[...] = jnp.dot(h, w2_ref[...],
                         preferred_element_type=jnp.float32) + b2_ref[...]


def _mlp_tc(x, W1p, s1, c1, W2, b22):
    vec = pl.BlockSpec((1, D), lambda i: (0, 0))
    mat = pl.BlockSpec((D, D), lambda i: (0, 0))
    return pl.pallas_call(
        _mlp_body,
        grid=(B // BLK,),
        in_specs=[pl.BlockSpec((BLK, D), lambda i: (i, 0)),
                  mat, vec, vec, mat, vec],
        out_specs=pl.BlockSpec((BLK, D), lambda i: (i, 0)),
        out_shape=jax.ShapeDtypeStruct((B, D), jnp.float32),
    )(x, W1p, s1, c1, W2, b22)


def kernel(classes, cond_drop_prob, emb_table, null_classes_emb,
           ln_gamma, ln_beta, W1, b1, W2, b2):
    W1g = ln_gamma[:, None] * W1
    s1 = jnp.sum(W1g, axis=0).reshape(1, D)
    c1 = (ln_beta @ W1 + b1).reshape(1, D)
    b22 = b2.reshape(1, D)
    W1p = W1g[_PERM, :]
    idx2d = classes.reshape(NW * NSTREAM, CH)
    table_i32 = lax.bitcast_convert_type(emb_table, jnp.int32)
    packed = _gather_sc(idx2d, table_i32)
    emb = lax.bitcast_convert_type(packed, jnp.bfloat16).reshape(B, D)
    return _mlp_tc(emb, W1p, s1, c1, W2, b22)


# speedup vs baseline: 1.0019x; 1.0019x over previous
import functools

import jax
import jax.numpy as jnp
import numpy as np
from jax import lax
from jax.experimental import pallas as pl
from jax.experimental.pallas import tpu as pltpu
from jax.experimental.pallas import tpu_sc as plsc

B = 16384
D = 128
NC = 2
NS = 16
NW = NC * NS
BPW = B // NW
CH = 128
NSTREAM = BPW // CH
BLK = 4096

_PERM = np.empty((D,), np.int64)
for _g in range(D // 32):
    for _i in range(16):
        _PERM[32 * _g + 2 * _i] = 32 * _g + _i
        _PERM[32 * _g + 2 * _i + 1] = 32 * _g + 16 + _i


def _gather_sc(idx2d, emb_table):
    mesh = plsc.VectorSubcoreMesh(core_axis_name="c", subcore_axis_name="s")

    @functools.partial(
        pl.kernel,
        mesh=mesh,
        out_type=jax.ShapeDtypeStruct((B, D // 2), jnp.int32),
        scratch_types=[
            pltpu.VMEM((NSTREAM, CH), jnp.int32),
            pltpu.VMEM((3, CH, D), jnp.int32),
            pltpu.VMEM((2, CH, D // 2), jnp.int32),
            pltpu.SemaphoreType.DMA,
            pltpu.SemaphoreType.DMA,
        ],
    )
    def k(idx_hbm, table_hbm, out_hbm, idx_v, rows_v, pw_v, sem, osem):
        wid = lax.axis_index("s") * NC + lax.axis_index("c")
        pltpu.sync_copy(idx_hbm.at[pl.ds(wid * NSTREAM, NSTREAM)], idx_v)

        def gather(j):
            return pltpu.async_copy(
                table_hbm.at[idx_v.at[j]], rows_v.at[j % 3], sem
            )

        copies = {0: gather(0), 1: gather(1)}
        outs = {}
        half = jnp.full((16,), 0x8000, jnp.int32)
        himask = jnp.full((16,), -65536, jnp.int32)
        for j in range(NSTREAM):
            copies[j].wait()
            if j + 2 < NSTREAM:
                copies[j + 2] = gather(j + 2)
            if j >= 2:
                outs[j - 2].wait()
            buf = j % 3
            obuf = j % 2

            @functools.partial(plsc.parallel_loop, 0, CH, unroll=8)
            def pack_row(r):
                for g in range(D // 32):
                    a = rows_v[buf, r, pl.ds(32 * g, 16)]
                    b = rows_v[buf, r, pl.ds(32 * g + 16, 16)]
                    w = lax.shift_right_logical(a + half, 16) | (
                        (b + half) & himask)
                    pw_v[obuf, r, pl.ds(16 * g, 16)] = w

            outs[j] = pltpu.async_copy(
                pw_v.at[obuf],
                out_hbm.at[pl.ds(wid * BPW + j * CH, CH)],
                osem,
            )
        outs[NSTREAM - 2].wait()
        outs[NSTREAM - 1].wait()

    return k(idx2d, emb_table)


def _mlp_body(x_ref, w1_ref, s1_ref, c1_ref, w2_ref, b2_ref, o_ref):
    x = x_ref[...].astype(jnp.float32)
    m = jnp.mean(x, axis=-1, keepdims=True)
    q = jnp.mean(x * x, axis=-1, keepdims=True)
    rstd = lax.rsqrt(q - m * m + 1e-5)
    p = jnp.dot(x, w1_ref[...], preferred_element_type=jnp.float32)
    h = rstd * (p - m * s1_ref[...]) + c1_ref[...]
    h = h * jax.nn.sigmoid(h)
    o_ref[...] = jnp.dot(h, w2_ref[...],
                         preferred_element_type=jnp.float32) + b2_ref[...]


def _mlp_tc(x, W1p, s1, c1, W2, b22):
    vec = pl.BlockSpec((1, D), lambda i: (0, 0))
    mat = pl.BlockSpec((D, D), lambda i: (0, 0))
    return pl.pallas_call(
        _mlp_body,
        grid=(B // BLK,),
        in_specs=[pl.BlockSpec((BLK, D), lambda i: (i, 0)),
                  mat, vec, vec, mat, vec],
        out_specs=pl.BlockSpec((BLK, D), lambda i: (i, 0)),
        out_shape=jax.ShapeDtypeStruct((B, D), jnp.float32),
    )(x, W1p, s1, c1, W2, b22)


def kernel(classes, cond_drop_prob, emb_table, null_classes_emb,
           ln_gamma, ln_beta, W1, b1, W2, b2):
    W1g = ln_gamma[:, None] * W1
    s1 = jnp.sum(W1g, axis=0).reshape(1, D)
    c1 = (ln_beta @ W1 + b1).reshape(1, D)
    b22 = b2.reshape(1, D)
    W1p = W1g[_PERM, :]
    idx2d = classes.reshape(NW * NSTREAM, CH)
    table_i32 = lax.bitcast_convert_type(emb_table, jnp.int32)
    packed = _gather_sc(idx2d, table_i32)
    emb = lax.bitcast_convert_type(packed, jnp.bfloat16).reshape(B, D)
    return _mlp_tc(emb, W1p, s1, c1, W2, b22)


# speedup vs baseline: 10.0906x; 10.0718x over previous
import functools

import jax
import jax.numpy as jnp
import numpy as np
from jax import lax
from jax.experimental import pallas as pl
from jax.experimental.pallas import tpu as pltpu
from jax.experimental.pallas import tpu_sc as plsc

B = 16384
D = 128
NC = 2
NS = 16
NW = NC * NS
BPW = B // NW
CH = 128
NSTREAM = BPW // CH
BLK = 4096

_PERM = np.empty((D,), np.int64)
for _c in range(D // 2):
    _PERM[_c] = 32 * (_c // 16) + _c % 16
    _PERM[D // 2 + _c] = 32 * (_c // 16) + 16 + _c % 16


def _gather_sc(idx2d, emb_table):
    mesh = plsc.VectorSubcoreMesh(core_axis_name="c", subcore_axis_name="s")

    @functools.partial(
        pl.kernel,
        mesh=mesh,
        out_type=jax.ShapeDtypeStruct((B, D // 2), jnp.int32),
        scratch_types=[
            pltpu.VMEM((NSTREAM, CH), jnp.int32),
            pltpu.VMEM((3, CH, D), jnp.float32),
            pltpu.VMEM((2, CH, D // 2), jnp.int32),
            pltpu.SemaphoreType.DMA,
            pltpu.SemaphoreType.DMA,
        ],
    )
    def k(idx_hbm, table_hbm, out_hbm, idx_v, rows_v, pw_v, sem, osem):
        wid = lax.axis_index("s") * NC + lax.axis_index("c")
        pltpu.sync_copy(idx_hbm.at[pl.ds(wid * NSTREAM, NSTREAM)], idx_v)

        def gather(j):
            return pltpu.async_copy(
                table_hbm.at[idx_v.at[j]], rows_v.at[j % 3], sem
            )

        copies = {0: gather(0), 1: gather(1)}
        outs = {}
        half = jnp.full((16,), 0x8000, jnp.int32)
        himask = jnp.full((16,), -65536, jnp.int32)
        for j in range(NSTREAM):
            copies[j].wait()
            if j + 2 < NSTREAM:
                copies[j + 2] = gather(j + 2)
            if j >= 2:
                outs[j - 2].wait()
            buf = j % 3
            obuf = j % 2

            def pack_row(r, _):
                for g in range(D // 32):
                    a = lax.bitcast_convert_type(
                        rows_v[buf, r, pl.ds(32 * g, 16)], jnp.int32)
                    b = lax.bitcast_convert_type(
                        rows_v[buf, r, pl.ds(32 * g + 16, 16)], jnp.int32)
                    w = lax.shift_right_logical(a + half, 16) | (
                        (b + half) & himask)
                    pw_v[obuf, r, pl.ds(16 * g, 16)] = w
                return ()

            lax.fori_loop(0, CH, pack_row, ())
            outs[j] = pltpu.async_copy(
                pw_v.at[obuf],
                out_hbm.at[pl.ds(wid * BPW + j * CH, CH)],
                osem,
            )
        outs[NSTREAM - 2].wait()
        outs[NSTREAM - 1].wait()

    return k(idx2d, emb_table)


def _mlp_body(x_ref, w1_ref, s1_ref, c1_ref, w2_ref, b2_ref, o_ref):
    xi = x_ref[...]
    lo = lax.bitcast_convert_type(xi << 16, jnp.float32)
    hi = lax.bitcast_convert_type(xi & jnp.int32(-65536), jnp.float32)
    x = jnp.concatenate([lo, hi], axis=1)
    m = jnp.mean(x, axis=-1, keepdims=True)
    q = jnp.mean(x * x, axis=-1, keepdims=True)
    rstd = lax.rsqrt(q - m * m + 1e-5)
    p = jnp.dot(x, w1_ref[...], preferred_element_type=jnp.float32)
    h = rstd * (p - m * s1_ref[...]) + c1_ref[...]
    h = h * jax.nn.sigmoid(h)
    o_ref[...] = jnp.dot(h, w2_ref[...],
                         preferred_element_type=jnp.float32) + b2_ref[...]


def _mlp_tc(packed, W1p, s1, c1, W2, b22):
    vec = pl.BlockSpec((1, D), lambda i: (0, 0))
    mat = pl.BlockSpec((D, D), lambda i: (0, 0))
    return pl.pallas_call(
        _mlp_body,
        grid=(B // BLK,),
        in_specs=[pl.BlockSpec((BLK, D // 2), lambda i: (i, 0)),
                  mat, vec, vec, mat, vec],
        out_specs=pl.BlockSpec((BLK, D), lambda i: (i, 0)),
        out_shape=jax.ShapeDtypeStruct((B, D), jnp.float32),
    )(packed, W1p, s1, c1, W2, b22)


def kernel(classes, cond_drop_prob, emb_table, null_classes_emb,
           ln_gamma, ln_beta, W1, b1, W2, b2):
    W1g = ln_gamma[:, None] * W1
    s1 = jnp.sum(W1g, axis=0).reshape(1, D)
    c1 = (ln_beta @ W1 + b1).reshape(1, D)
    b22 = b2.reshape(1, D)
    W1p = W1g[_PERM, :]
    idx2d = classes.reshape(NW * NSTREAM, CH)
    packed = _gather_sc(idx2d, emb_table)
    return _mlp_tc(packed, W1p, s1, c1, W2, b22)


# speedup vs baseline: 10.8391x; 1.0742x over previous
import functools

import jax
import jax.numpy as jnp
from jax import lax
from jax.experimental import pallas as pl
from jax.experimental.pallas import tpu as pltpu
from jax.experimental.pallas import tpu_sc as plsc

B = 16384
D = 128
NC = 2
NS = 16
NW = NC * NS
HB = B // 2
BPW = HB // NW
CH = 128
NCHUNK = BPW // CH
BLK = 2048


def _gather_sc(idx4, emb_table):
    mesh = plsc.VectorSubcoreMesh(core_axis_name="c", subcore_axis_name="s")

    @functools.partial(
        pl.kernel,
        mesh=mesh,
        out_type=jax.ShapeDtypeStruct((HB, D), jnp.int32),
        scratch_types=[
            pltpu.VMEM((2 * NCHUNK, CH), jnp.int32),
            pltpu.VMEM((2 * NCHUNK, CH, D), jnp.float32),
            pltpu.VMEM((CH, D), jnp.int32),
            pltpu.SemaphoreType.DMA,
            pltpu.SemaphoreType.DMA,
        ],
    )
    def k(idx_hbm, table_hbm, out_hbm, idx_v, rows_v, pw_v, sem, osem):
        wid = lax.axis_index("s") * NC + lax.axis_index("c")
        pltpu.sync_copy(
            idx_hbm.at[pl.ds(wid * 2 * NCHUNK, 2 * NCHUNK)], idx_v)
        copies = [
            pltpu.async_copy(table_hbm.at[idx_v.at[s]], rows_v.at[s], sem)
            for s in range(2 * NCHUNK)
        ]
        half = jnp.full((16,), 0x8000, jnp.int32)
        himask = jnp.full((16,), -65536, jnp.int32)
        out_copy = None
        for c in range(NCHUNK):
            copies[2 * c].wait()
            copies[2 * c + 1].wait()
            if out_copy is not None:
                out_copy.wait()

            def pack_row(r, _):
                for g in range(D // 16):
                    a = lax.bitcast_convert_type(
                        rows_v[2 * c, r, pl.ds(16 * g, 16)], jnp.int32)
                    b = lax.bitcast_convert_type(
                        rows_v[2 * c + 1, r, pl.ds(16 * g, 16)], jnp.int32)
                    w = lax.shift_right_logical(a + half, 16) | (
                        (b + half) & himask)
                    pw_v[r, pl.ds(16 * g, 16)] = w
                return ()

            lax.fori_loop(0, CH, pack_row, ())
            out_copy = pltpu.async_copy(
                pw_v,
                out_hbm.at[pl.ds(wid * BPW + c * CH, CH)],
                osem,
            )
        out_copy.wait()

    return k(idx4, emb_table)


def _mlp_body(x_ref, w1_ref, s1_ref, c1_ref, w2_ref, b2_ref, o_ref):
    xi = x_ref[...]
    lo = lax.bitcast_convert_type(xi << 16, jnp.float32)
    hi = lax.bitcast_convert_type(xi & jnp.int32(-65536), jnp.float32)
    for sel, x in ((0, lo), (1, hi)):
        m = jnp.mean(x, axis=-1, keepdims=True)
        q = jnp.mean(x * x, axis=-1, keepdims=True)
        rstd = lax.rsqrt(q - m * m + 1e-5)
        p = jnp.dot(x, w1_ref[...], preferred_element_type=jnp.float32)
        h = rstd * (p - m * s1_ref[...]) + c1_ref[...]
        h = h * jax.nn.sigmoid(h)
        o_ref[sel] = jnp.dot(h, w2_ref[...],
                             preferred_element_type=jnp.float32) + b2_ref[...]


def _mlp_tc(packed, W1g, s1, c1, W2, b22):
    vec = pl.BlockSpec((1, D), lambda i: (0, 0))
    mat = pl.BlockSpec((D, D), lambda i: (0, 0))
    return pl.pallas_call(
        _mlp_body,
        grid=(HB // BLK,),
        in_specs=[pl.BlockSpec((BLK, D), lambda i: (i, 0)),
                  mat, vec, vec, mat, vec],
        out_specs=pl.BlockSpec((2, BLK, D), lambda i: (0, i, 0)),
        out_shape=jax.ShapeDtypeStruct((2, HB, D), jnp.float32),
    )(packed, W1g, s1, c1, W2, b22)


def kernel(classes, cond_drop_prob, emb_table, null_classes_emb,
           ln_gamma, ln_beta, W1, b1, W2, b2):
    W1g = ln_gamma[:, None] * W1
    s1 = jnp.sum(W1g, axis=0).reshape(1, D)
    c1 = (ln_beta @ W1 + b1).reshape(1, D)
    b22 = b2.reshape(1, D)
    idx4 = classes.reshape(2, NW, NCHUNK, CH).transpose(1, 2, 0, 3)
    idx4 = idx4.reshape(NW * 2 * NCHUNK, CH)
    packed = _gather_sc(idx4, emb_table)
    out3 = _mlp_tc(packed, W1g, s1, c1, W2, b22)
    return out3.reshape(B, D)
